# Initial kernel scaffold; baseline (speedup 1.0000x reference)
import jax
import jax.numpy as jnp
from jax.experimental import pallas as pl


def kernel(x, edge_attr, edge_index, params):
    raise NotImplementedError("write your pallas kernel here")


# speedup vs baseline: 1.7338x; 1.7338x over previous
import functools

import jax
import jax.numpy as jnp
from jax import lax
from jax.experimental import pallas as pl
from jax.experimental.pallas import tpu as pltpu
from jax.experimental.pallas import tpu_sc as plsc

N = 10000
E = 160000
D = 128
TWO_D = 2 * D
EP = E + N
NW = 32
CHUNK = 128
EPW = 5376
EPAD = EPW * NW
NCHUNK = EPW // CHUNK
NACC = 10240
NTS = NACC // 16
BE = 512
BN = 1000


def _tables_body(x_ref, wd_ref, ws_ref, td_ref, ts_ref):
    xb = x_ref[...]
    td_ref[...] = jnp.dot(xb, wd_ref[...], preferred_element_type=jnp.float32)
    ts_ref[...] = jnp.dot(xb, ws_ref[...], preferred_element_type=jnp.float32)


def _tables(x, wd, ws):
    return pl.pallas_call(
        _tables_body,
        grid=(N // BN,),
        in_specs=[
            pl.BlockSpec((BN, D), lambda i: (i, 0)),
            pl.BlockSpec((D, TWO_D), lambda i: (0, 0)),
            pl.BlockSpec((D, TWO_D), lambda i: (0, 0)),
        ],
        out_specs=[
            pl.BlockSpec((BN, TWO_D), lambda i: (i, 0)),
            pl.BlockSpec((BN, TWO_D), lambda i: (i, 0)),
        ],
        out_shape=[jax.ShapeDtypeStruct((N, TWO_D), jnp.float32)] * 2,
    )(x, wd, ws)


def _edge_body(ai_ref, aj_ref, ea_ref, we_ref, w2e_ref, wme_ref, v2_ref,
               bias_ref, ne_ref, msg_ref):
    i = pl.program_id(0)
    ai = ai_ref[...]
    aj = aj_ref[...]
    b1e = bias_ref[0:1, :]
    b2e = bias_ref[1:2, :]
    c1 = bias_ref[2:3, :]
    c2 = bias_ref[3:4, :]
    pre_e = (ai[:, :D] + aj[:, :D] + b1e
             + jnp.dot(ea_ref[...], we_ref[...],
                       preferred_element_type=jnp.float32))
    h = jnp.maximum(pre_e, 0.0)
    ne = jnp.dot(h, w2e_ref[...], preferred_element_type=jnp.float32) + b2e
    pre_m = (ai[:, D:] + aj[:, D:] + c1
             + jnp.dot(ne, wme_ref[...], preferred_element_type=jnp.float32))
    h2 = jnp.maximum(pre_m, 0.0)
    msg = jnp.dot(h2, v2_ref[...], preferred_element_type=jnp.float32) + c2
    ne_ref[...] = ne
    rows = i * BE + lax.broadcasted_iota(jnp.int32, (BE, 1), 0)
    msg_ref[...] = jnp.where(rows < EP, msg, 0.0)


def _edge_mlps(ai, aj, ea, we, w2e, wme, v2, bias):
    d_e = ea.shape[1]
    return pl.pallas_call(
        _edge_body,
        grid=(EPAD // BE,),
        in_specs=[
            pl.BlockSpec((BE, TWO_D), lambda i: (i, 0)),
            pl.BlockSpec((BE, TWO_D), lambda i: (i, 0)),
            pl.BlockSpec((BE, d_e), lambda i: (i, 0)),
            pl.BlockSpec((d_e, D), lambda i: (0, 0)),
            pl.BlockSpec((D, D), lambda i: (0, 0)),
            pl.BlockSpec((D, D), lambda i: (0, 0)),
            pl.BlockSpec((D, D), lambda i: (0, 0)),
            pl.BlockSpec((8, D), lambda i: (0, 0)),
        ],
        out_specs=[
            pl.BlockSpec((BE, D), lambda i: (i, 0)),
            pl.BlockSpec((BE, D), lambda i: (i, 0)),
        ],
        out_shape=[jax.ShapeDtypeStruct((EPAD, D), jnp.float32)] * 2,
    )(ai, aj, ea, we, w2e, wme, v2, bias)


def _node_body_tables(x_ref, p_ref, u1x_ref, u1a_ref, u2_ref, bias_ref,
                      wd_ref, ws_ref, xo_ref, td_ref, ts_ref):
    aggr = p_ref[0] + p_ref[1]
    d1 = bias_ref[0:1, :]
    d2 = bias_ref[1:2, :]
    pre = (jnp.dot(x_ref[...], u1x_ref[...], preferred_element_type=jnp.float32)
           + jnp.dot(aggr, u1a_ref[...], preferred_element_type=jnp.float32)
           + d1)
    xn = jnp.dot(jnp.maximum(pre, 0.0), u2_ref[...],
                 preferred_element_type=jnp.float32) + d2
    xo_ref[...] = xn
    td_ref[...] = jnp.dot(xn, wd_ref[...], preferred_element_type=jnp.float32)
    ts_ref[...] = jnp.dot(xn, ws_ref[...], preferred_element_type=jnp.float32)


def _node_body(x_ref, p_ref, u1x_ref, u1a_ref, u2_ref, bias_ref, xo_ref):
    aggr = p_ref[0] + p_ref[1]
    d1 = bias_ref[0:1, :]
    d2 = bias_ref[1:2, :]
    pre = (jnp.dot(x_ref[...], u1x_ref[...], preferred_element_type=jnp.float32)
           + jnp.dot(aggr, u1a_ref[...], preferred_element_type=jnp.float32)
           + d1)
    xo_ref[...] = jnp.dot(jnp.maximum(pre, 0.0), u2_ref[...],
                          preferred_element_type=jnp.float32) + d2


def _node_update(x, partials, u1x, u1a, u2, bias, wd=None, ws=None):
    mat = lambda i: (0, 0)
    in_specs = [
        pl.BlockSpec((BN, D), lambda i: (i, 0)),
        pl.BlockSpec((2, BN, D), lambda i: (0, i, 0)),
        pl.BlockSpec((D, D), mat),
        pl.BlockSpec((D, D), mat),
        pl.BlockSpec((D, D), mat),
        pl.BlockSpec((8, D), mat),
    ]
    if wd is None:
        return pl.pallas_call(
            _node_body,
            grid=(N // BN,),
            in_specs=in_specs,
            out_specs=pl.BlockSpec((BN, D), lambda i: (i, 0)),
            out_shape=jax.ShapeDtypeStruct((N, D), jnp.float32),
        )(x, partials, u1x, u1a, u2, bias)
    in_specs += [pl.BlockSpec((D, TWO_D), mat), pl.BlockSpec((D, TWO_D), mat)]
    return pl.pallas_call(
        _node_body_tables,
        grid=(N // BN,),
        in_specs=in_specs,
        out_specs=[
            pl.BlockSpec((BN, D), lambda i: (i, 0)),
            pl.BlockSpec((BN, TWO_D), lambda i: (i, 0)),
            pl.BlockSpec((BN, TWO_D), lambda i: (i, 0)),
        ],
        out_shape=[
            jax.ShapeDtypeStruct((N, D), jnp.float32),
            jax.ShapeDtypeStruct((N, TWO_D), jnp.float32),
            jax.ShapeDtypeStruct((N, TWO_D), jnp.float32),
        ],
    )(x, partials, u1x, u1a, u2, bias, wd, ws)


def _gather_kernel():
    mesh = plsc.VectorSubcoreMesh(core_axis_name="c", subcore_axis_name="s")

    def body(td_hbm, ts_hbm, dst_hbm, src_hbm, ai_hbm, aj_hbm,
             idxd, idxs, rowsd, rowss, semd, sems):
        wid = lax.axis_index("s") * 2 + lax.axis_index("c")
        base = wid * EPW

        @pl.loop(0, NCHUNK)
        def _(ci):
            eb = base + ci * CHUNK
            pltpu.sync_copy(dst_hbm.at[pl.ds(eb, CHUNK)], idxd)
            pltpu.sync_copy(src_hbm.at[pl.ds(eb, CHUNK)], idxs)
            cd = pltpu.async_copy(td_hbm.at[idxd], rowsd, semd)
            cs = pltpu.async_copy(ts_hbm.at[idxs], rowss, sems)
            cd.wait()
            cs.wait()
            pltpu.sync_copy(rowsd, ai_hbm.at[pl.ds(eb, CHUNK)])
            pltpu.sync_copy(rowss, aj_hbm.at[pl.ds(eb, CHUNK)])

    return pl.kernel(
        body,
        out_type=[jax.ShapeDtypeStruct((EPAD, TWO_D), jnp.float32)] * 2,
        mesh=mesh,
        scratch_types=[
            pltpu.VMEM((CHUNK,), jnp.int32),
            pltpu.VMEM((CHUNK,), jnp.int32),
            pltpu.VMEM((CHUNK, TWO_D), jnp.float32),
            pltpu.VMEM((CHUNK, TWO_D), jnp.float32),
            pltpu.SemaphoreType.DMA,
            pltpu.SemaphoreType.DMA,
        ],
    )


def _scatter_kernel():
    mesh = plsc.VectorSubcoreMesh(core_axis_name="c", subcore_axis_name="s")

    def body(msg_hbm, dst_hbm, out_hbm, idx, mbuf, acc):
        cid = lax.axis_index("c")
        sid = lax.axis_index("s")
        wid = sid * 2 + cid

        @pl.loop(0, CHUNK)
        def _(i):
            for j in range(D // 16):
                mbuf[i, pl.ds(j * 16, 16)] = jnp.zeros((16,), jnp.float32)

        rbase = sid * NTS
        for k in range(NTS // CHUNK):
            pltpu.sync_copy(mbuf, acc.at[pl.ds(rbase + k * CHUNK, CHUNK)])
        plsc.subcore_barrier()

        base = wid * EPW

        @pl.loop(0, NCHUNK)
        def _(ci):
            eb = base + ci * CHUNK
            pltpu.sync_copy(dst_hbm.at[pl.ds(eb, CHUNK)], idx)
            pltpu.sync_copy(msg_hbm.at[pl.ds(eb, CHUNK)], mbuf)
            pltpu.sync_copy(mbuf, acc.at[idx], add=True)

        plsc.subcore_barrier()
        pltpu.sync_copy(acc.at[pl.ds(rbase, NTS)],
                        out_hbm.at[cid, pl.ds(rbase, NTS)])

    return pl.kernel(
        body,
        out_type=jax.ShapeDtypeStruct((2, NACC, D), jnp.float32),
        mesh=mesh,
        scratch_types=[
            pltpu.VMEM((CHUNK,), jnp.int32),
            pltpu.VMEM((CHUNK, D), jnp.float32),
            pltpu.VMEM_SHARED((NACC, D), jnp.float32),
        ],
    )


def _prep_layer(p):
    en, mm = p["en"], p["mm"]
    wd = jnp.concatenate([en["W1"][:D], mm["W1"][:D]], axis=1)
    ws = jnp.concatenate([en["W1"][D:TWO_D], mm["W1"][D:TWO_D]], axis=1)
    we = en["W1"][TWO_D:]
    wme = mm["W1"][TWO_D:]
    bias = (jnp.zeros((8, D), jnp.float32)
            .at[0].set(en["b1"]).at[1].set(en["b2"])
            .at[2].set(mm["b1"]).at[3].set(mm["b2"]))
    return wd, ws, we, en["W2"], wme, mm["W2"], bias


def _prep_node(p):
    nu = p["nu"]
    bias = (jnp.zeros((8, D), jnp.float32)
            .at[0].set(nu["b1"]).at[1].set(nu["b2"]))
    return nu["W1"][:D], nu["W1"][D:], nu["W2"], bias


@jax.jit
def kernel(x, edge_attr, edge_index, params):
    ei = edge_index.astype(jnp.int32)
    loops = jnp.arange(N, dtype=jnp.int32)
    pad = jnp.zeros((EPAD - EP,), jnp.int32)
    src = jnp.concatenate([ei[0], loops, pad])
    dst = jnp.concatenate([ei[1], loops, pad])
    ea0 = jnp.concatenate(
        [edge_attr, jnp.zeros((EPAD - E, edge_attr.shape[1]), jnp.float32)])

    wd0, ws0, we0, w2e0, wme0, v20, be0 = _prep_layer(params["l0"])
    wd1, ws1, we1, w2e1, wme1, v21, be1 = _prep_layer(params["l1"])
    u1x0, u1a0, u20, bn0 = _prep_node(params["l0"])
    u1x1, u1a1, u21, bn1 = _prep_node(params["l1"])

    gather = _gather_kernel()
    scatter = _scatter_kernel()

    td0, ts0 = _tables(x, wd0, ws0)
    ai, aj = gather(td0, ts0, dst, src)
    ne0, msg0 = _edge_mlps(ai, aj, ea0, we0, w2e0, wme0, v20, be0)
    part0 = scatter(msg0, dst)
    x1, td1, ts1 = _node_update(x, part0, u1x0, u1a0, u20, bn0, wd1, ws1)

    ai, aj = gather(td1, ts1, dst, src)
    ne1, msg1 = _edge_mlps(ai, aj, ne0, we1, w2e1, wme1, v21, be1)
    part1 = scatter(msg1, dst)
    x2 = _node_update(x1, part1, u1x1, u1a1, u21, bn1)

    return (x2, ne1[:EP])


# speedup vs baseline: 1.9878x; 1.1465x over previous
import functools

import jax
import jax.numpy as jnp
from jax import lax
from jax.experimental import pallas as pl
from jax.experimental.pallas import tpu as pltpu
from jax.experimental.pallas import tpu_sc as plsc

N = 10000
E = 160000
D = 128
TWO_D = 2 * D
EP = E + N
NW = 32
CHUNK = 112
EPW = 5376
EPAD = EPW * NW
NCHUNK = EPW // CHUNK
NACC = 10240
NTS = NACC // 16
BE = 512
BN = 1000


def _tables_body(x_ref, wd_ref, ws_ref, td_ref, ts_ref):
    xb = x_ref[...]
    td_ref[...] = jnp.dot(xb, wd_ref[...], preferred_element_type=jnp.float32)
    ts_ref[...] = jnp.dot(xb, ws_ref[...], preferred_element_type=jnp.float32)


def _tables(x, wd, ws):
    return pl.pallas_call(
        _tables_body,
        grid=(N // BN,),
        in_specs=[
            pl.BlockSpec((BN, D), lambda i: (i, 0)),
            pl.BlockSpec((D, TWO_D), lambda i: (0, 0)),
            pl.BlockSpec((D, TWO_D), lambda i: (0, 0)),
        ],
        out_specs=[
            pl.BlockSpec((BN, TWO_D), lambda i: (i, 0)),
            pl.BlockSpec((BN, TWO_D), lambda i: (i, 0)),
        ],
        out_shape=[jax.ShapeDtypeStruct((N, TWO_D), jnp.float32)] * 2,
    )(x, wd, ws)


def _edge_body(ai_ref, aj_ref, ea_ref, we_ref, w2e_ref, wme_ref, v2_ref,
               bias_ref, ne_ref, msg_ref):
    i = pl.program_id(0)
    ai = ai_ref[...]
    aj = aj_ref[...]
    b1e = bias_ref[0:1, :]
    b2e = bias_ref[1:2, :]
    c1 = bias_ref[2:3, :]
    c2 = bias_ref[3:4, :]
    pre_e = (ai[:, :D] + aj[:, :D] + b1e
             + jnp.dot(ea_ref[...], we_ref[...],
                       preferred_element_type=jnp.float32))
    h = jnp.maximum(pre_e, 0.0)
    ne = jnp.dot(h, w2e_ref[...], preferred_element_type=jnp.float32) + b2e
    pre_m = (ai[:, D:] + aj[:, D:] + c1
             + jnp.dot(ne, wme_ref[...], preferred_element_type=jnp.float32))
    h2 = jnp.maximum(pre_m, 0.0)
    msg = jnp.dot(h2, v2_ref[...], preferred_element_type=jnp.float32) + c2
    ne_ref[...] = ne
    rows = i * BE + lax.broadcasted_iota(jnp.int32, (BE, 1), 0)
    msg_ref[...] = jnp.where(rows < EP, msg, 0.0)


def _edge_mlps(ai, aj, ea, we, w2e, wme, v2, bias):
    d_e = ea.shape[1]
    return pl.pallas_call(
        _edge_body,
        grid=(EPAD // BE,),
        in_specs=[
            pl.BlockSpec((BE, TWO_D), lambda i: (i, 0)),
            pl.BlockSpec((BE, TWO_D), lambda i: (i, 0)),
            pl.BlockSpec((BE, d_e), lambda i: (i, 0)),
            pl.BlockSpec((d_e, D), lambda i: (0, 0)),
            pl.BlockSpec((D, D), lambda i: (0, 0)),
            pl.BlockSpec((D, D), lambda i: (0, 0)),
            pl.BlockSpec((D, D), lambda i: (0, 0)),
            pl.BlockSpec((8, D), lambda i: (0, 0)),
        ],
        out_specs=[
            pl.BlockSpec((BE, D), lambda i: (i, 0)),
            pl.BlockSpec((BE, D), lambda i: (i, 0)),
        ],
        out_shape=[jax.ShapeDtypeStruct((EPAD, D), jnp.float32)] * 2,
    )(ai, aj, ea, we, w2e, wme, v2, bias)


def _node_body_tables(x_ref, p_ref, u1x_ref, u1a_ref, u2_ref, bias_ref,
                      wd_ref, ws_ref, xo_ref, td_ref, ts_ref):
    aggr = p_ref[0] + p_ref[1]
    d1 = bias_ref[0:1, :]
    d2 = bias_ref[1:2, :]
    pre = (jnp.dot(x_ref[...], u1x_ref[...], preferred_element_type=jnp.float32)
           + jnp.dot(aggr, u1a_ref[...], preferred_element_type=jnp.float32)
           + d1)
    xn = jnp.dot(jnp.maximum(pre, 0.0), u2_ref[...],
                 preferred_element_type=jnp.float32) + d2
    xo_ref[...] = xn
    td_ref[...] = jnp.dot(xn, wd_ref[...], preferred_element_type=jnp.float32)
    ts_ref[...] = jnp.dot(xn, ws_ref[...], preferred_element_type=jnp.float32)


def _node_body(x_ref, p_ref, u1x_ref, u1a_ref, u2_ref, bias_ref, xo_ref):
    aggr = p_ref[0] + p_ref[1]
    d1 = bias_ref[0:1, :]
    d2 = bias_ref[1:2, :]
    pre = (jnp.dot(x_ref[...], u1x_ref[...], preferred_element_type=jnp.float32)
           + jnp.dot(aggr, u1a_ref[...], preferred_element_type=jnp.float32)
           + d1)
    xo_ref[...] = jnp.dot(jnp.maximum(pre, 0.0), u2_ref[...],
                          preferred_element_type=jnp.float32) + d2


def _node_update(x, partials, u1x, u1a, u2, bias, wd=None, ws=None):
    mat = lambda i: (0, 0)
    in_specs = [
        pl.BlockSpec((BN, D), lambda i: (i, 0)),
        pl.BlockSpec((2, BN, D), lambda i: (0, i, 0)),
        pl.BlockSpec((D, D), mat),
        pl.BlockSpec((D, D), mat),
        pl.BlockSpec((D, D), mat),
        pl.BlockSpec((8, D), mat),
    ]
    if wd is None:
        return pl.pallas_call(
            _node_body,
            grid=(N // BN,),
            in_specs=in_specs,
            out_specs=pl.BlockSpec((BN, D), lambda i: (i, 0)),
            out_shape=jax.ShapeDtypeStruct((N, D), jnp.float32),
        )(x, partials, u1x, u1a, u2, bias)
    in_specs += [pl.BlockSpec((D, TWO_D), mat), pl.BlockSpec((D, TWO_D), mat)]
    return pl.pallas_call(
        _node_body_tables,
        grid=(N // BN,),
        in_specs=in_specs,
        out_specs=[
            pl.BlockSpec((BN, D), lambda i: (i, 0)),
            pl.BlockSpec((BN, TWO_D), lambda i: (i, 0)),
            pl.BlockSpec((BN, TWO_D), lambda i: (i, 0)),
        ],
        out_shape=[
            jax.ShapeDtypeStruct((N, D), jnp.float32),
            jax.ShapeDtypeStruct((N, TWO_D), jnp.float32),
            jax.ShapeDtypeStruct((N, TWO_D), jnp.float32),
        ],
    )(x, partials, u1x, u1a, u2, bias, wd, ws)


def _gather_kernel():
    mesh = plsc.VectorSubcoreMesh(core_axis_name="c", subcore_axis_name="s")

    def body(td_hbm, ts_hbm, dst_hbm, src_hbm, ai_hbm, aj_hbm,
             idxd, idxs, rowsd0, rowss0, rowsd1, rowss1,
             gd0, gs0, gd1, gs1):
        wid = lax.axis_index("s") * 2 + lax.axis_index("c")
        base = wid * EPW

        pltpu.sync_copy(dst_hbm.at[wid], idxd)
        pltpu.sync_copy(src_hbm.at[wid], idxs)

        def start(c, rd, rs, sd, ss):
            pltpu.async_copy(td_hbm.at[idxd.at[c]], rd, sd)
            pltpu.async_copy(ts_hbm.at[idxs.at[c]], rs, ss)

        def drain(c, rd, rs, sd, ss):
            pltpu.make_async_copy(td_hbm.at[idxd.at[c]], rd, sd).wait()
            pltpu.make_async_copy(ts_hbm.at[idxs.at[c]], rs, ss).wait()
            eb = base + c * CHUNK
            pltpu.sync_copy(rd, ai_hbm.at[pl.ds(eb, CHUNK)])
            pltpu.sync_copy(rs, aj_hbm.at[pl.ds(eb, CHUNK)])

        start(0, rowsd0, rowss0, gd0, gs0)

        @pl.loop(0, NCHUNK // 2)
        def _(g):
            c = 2 * g
            start(c + 1, rowsd1, rowss1, gd1, gs1)
            drain(c, rowsd0, rowss0, gd0, gs0)

            @pl.when(c + 2 < NCHUNK)
            def _():
                start(c + 2, rowsd0, rowss0, gd0, gs0)

            drain(c + 1, rowsd1, rowss1, gd1, gs1)

    return pl.kernel(
        body,
        out_type=[jax.ShapeDtypeStruct((EPAD, TWO_D), jnp.float32)] * 2,
        mesh=mesh,
        scratch_types=[
            pltpu.VMEM((NCHUNK, CHUNK), jnp.int32),
            pltpu.VMEM((NCHUNK, CHUNK), jnp.int32),
            pltpu.VMEM((CHUNK, TWO_D), jnp.float32),
            pltpu.VMEM((CHUNK, TWO_D), jnp.float32),
            pltpu.VMEM((CHUNK, TWO_D), jnp.float32),
            pltpu.VMEM((CHUNK, TWO_D), jnp.float32),
            pltpu.SemaphoreType.DMA,
            pltpu.SemaphoreType.DMA,
            pltpu.SemaphoreType.DMA,
            pltpu.SemaphoreType.DMA,
        ],
    )


def _scatter_kernel():
    mesh = plsc.VectorSubcoreMesh(core_axis_name="c", subcore_axis_name="s")

    def body(msg_hbm, dst_hbm, out_hbm, idx, mbuf0, mbuf1, acc, ls0, ls1):
        cid = lax.axis_index("c")
        sid = lax.axis_index("s")
        wid = sid * 2 + cid
        base = wid * EPW

        pltpu.sync_copy(dst_hbm.at[wid], idx)

        @pl.loop(0, CHUNK)
        def _(i):
            for j in range(D // 16):
                mbuf0[i, pl.ds(j * 16, 16)] = jnp.zeros((16,), jnp.float32)

        rbase = sid * NTS
        for k in range(NTS // CHUNK):
            pltpu.sync_copy(mbuf0, acc.at[pl.ds(rbase + k * CHUNK, CHUNK)])
        rem = NTS - (NTS // CHUNK) * CHUNK
        if rem:
            pltpu.sync_copy(mbuf0.at[pl.ds(0, rem)],
                            acc.at[pl.ds(rbase + NTS - rem, rem)])
        plsc.subcore_barrier()

        def start(c, mb, sem):
            pltpu.async_copy(msg_hbm.at[pl.ds(base + c * CHUNK, CHUNK)],
                             mb, sem)

        def drain(c, mb, sem):
            pltpu.make_async_copy(msg_hbm.at[pl.ds(base + c * CHUNK, CHUNK)],
                                  mb, sem).wait()
            pltpu.sync_copy(mb, acc.at[idx.at[c]], add=True)

        start(0, mbuf0, ls0)

        @pl.loop(0, NCHUNK // 2)
        def _(g):
            c = 2 * g
            start(c + 1, mbuf1, ls1)
            drain(c, mbuf0, ls0)

            @pl.when(c + 2 < NCHUNK)
            def _():
                start(c + 2, mbuf0, ls0)

            drain(c + 1, mbuf1, ls1)

        plsc.subcore_barrier()
        pltpu.sync_copy(acc.at[pl.ds(rbase, NTS)],
                        out_hbm.at[cid, pl.ds(rbase, NTS)])

    return pl.kernel(
        body,
        out_type=jax.ShapeDtypeStruct((2, NACC, D), jnp.float32),
        mesh=mesh,
        scratch_types=[
            pltpu.VMEM((NCHUNK, CHUNK), jnp.int32),
            pltpu.VMEM((CHUNK, D), jnp.float32),
            pltpu.VMEM((CHUNK, D), jnp.float32),
            pltpu.VMEM_SHARED((NACC, D), jnp.float32),
            pltpu.SemaphoreType.DMA,
            pltpu.SemaphoreType.DMA,
        ],
    )


def _prep_layer(p):
    en, mm = p["en"], p["mm"]
    wd = jnp.concatenate([en["W1"][:D], mm["W1"][:D]], axis=1)
    ws = jnp.concatenate([en["W1"][D:TWO_D], mm["W1"][D:TWO_D]], axis=1)
    we = en["W1"][TWO_D:]
    wme = mm["W1"][TWO_D:]
    bias = (jnp.zeros((8, D), jnp.float32)
            .at[0].set(en["b1"]).at[1].set(en["b2"])
            .at[2].set(mm["b1"]).at[3].set(mm["b2"]))
    return wd, ws, we, en["W2"], wme, mm["W2"], bias


def _prep_node(p):
    nu = p["nu"]
    bias = (jnp.zeros((8, D), jnp.float32)
            .at[0].set(nu["b1"]).at[1].set(nu["b2"]))
    return nu["W1"][:D], nu["W1"][D:], nu["W2"], bias


@jax.jit
def kernel(x, edge_attr, edge_index, params):
    ei = edge_index.astype(jnp.int32)
    loops = jnp.arange(N, dtype=jnp.int32)
    pad = jnp.zeros((EPAD - EP,), jnp.int32)
    src = jnp.concatenate([ei[0], loops, pad]).reshape(NW, NCHUNK, CHUNK)
    dst = jnp.concatenate([ei[1], loops, pad]).reshape(NW, NCHUNK, CHUNK)
    ea0 = jnp.concatenate(
        [edge_attr, jnp.zeros((EPAD - E, edge_attr.shape[1]), jnp.float32)])

    wd0, ws0, we0, w2e0, wme0, v20, be0 = _prep_layer(params["l0"])
    wd1, ws1, we1, w2e1, wme1, v21, be1 = _prep_layer(params["l1"])
    u1x0, u1a0, u20, bn0 = _prep_node(params["l0"])
    u1x1, u1a1, u21, bn1 = _prep_node(params["l1"])

    gather = _gather_kernel()
    scatter = _scatter_kernel()

    td0, ts0 = _tables(x, wd0, ws0)
    ai, aj = gather(td0, ts0, dst, src)
    ne0, msg0 = _edge_mlps(ai, aj, ea0, we0, w2e0, wme0, v20, be0)
    part0 = scatter(msg0, dst)
    x1, td1, ts1 = _node_update(x, part0, u1x0, u1a0, u20, bn0, wd1, ws1)

    ai, aj = gather(td1, ts1, dst, src)
    ne1, msg1 = _edge_mlps(ai, aj, ne0, we1, w2e1, wme1, v21, be1)
    part1 = scatter(msg1, dst)
    x2 = _node_update(x1, part1, u1x1, u1a1, u21, bn1)

    return (x2, ne1[:EP])


# speedup vs baseline: 2.5142x; 1.2648x over previous
import functools

import jax
import jax.numpy as jnp
from jax import lax
from jax.experimental import pallas as pl
from jax.experimental.pallas import tpu as pltpu
from jax.experimental.pallas import tpu_sc as plsc

N = 10000
E = 160000
D = 128
TWO_D = 2 * D
EP = E + N
NW = 32
CHUNK = 128
EPW = 5376
EPAD = EPW * NW
NCHUNK = EPW // CHUNK
NACC = 10240
NTS = NACC // 16
BE = 512
BN = 1000


def _pack2(a, b):
    ai = lax.bitcast_convert_type(
        a.astype(jnp.bfloat16).astype(jnp.float32), jnp.int32)
    bi = lax.bitcast_convert_type(
        b.astype(jnp.bfloat16).astype(jnp.float32), jnp.int32)
    return lax.shift_right_logical(ai, 16) | bi


def _unpack2(p):
    lo = lax.bitcast_convert_type(lax.shift_left(p, 16), jnp.float32)
    hi = lax.bitcast_convert_type(p & jnp.int32(-65536), jnp.float32)
    return (lo, hi)


def _tables_body(x_ref, wd_ref, ws_ref, td_ref, ts_ref):
    xb = x_ref[...]
    td = jnp.dot(xb, wd_ref[...], preferred_element_type=jnp.float32)
    ts = jnp.dot(xb, ws_ref[...], preferred_element_type=jnp.float32)
    td_ref[...] = _pack2(td[:, :D], td[:, D:])
    ts_ref[...] = _pack2(ts[:, :D], ts[:, D:])


def _tables(x, wd, ws):
    return pl.pallas_call(
        _tables_body,
        grid=(N // BN,),
        in_specs=[
            pl.BlockSpec((BN, D), lambda i: (i, 0)),
            pl.BlockSpec((D, TWO_D), lambda i: (0, 0)),
            pl.BlockSpec((D, TWO_D), lambda i: (0, 0)),
        ],
        out_specs=[
            pl.BlockSpec((BN, D), lambda i: (i, 0)),
            pl.BlockSpec((BN, D), lambda i: (i, 0)),
        ],
        out_shape=[jax.ShapeDtypeStruct((N, D), jnp.int32)] * 2,
    )(x, wd, ws)


def _edge_body(ai_ref, aj_ref, ea_ref, we_ref, w2e_ref, wme_ref, v2_ref,
               bias_ref, ne_ref, msg_ref):
    i = pl.program_id(0)
    b1e = bias_ref[0:1, :]
    b2e = bias_ref[1:2, :]
    c1 = bias_ref[2:3, :]
    c2 = bias_ref[3:4, :]
    ai_e, ai_m = _unpack2(ai_ref[...])
    aj_e, aj_m = _unpack2(aj_ref[...])
    pre_e = (ai_e + aj_e + b1e
             + jnp.dot(ea_ref[...], we_ref[...],
                       preferred_element_type=jnp.float32))
    h = jnp.maximum(pre_e, 0.0)
    ne = jnp.dot(h, w2e_ref[...], preferred_element_type=jnp.float32) + b2e
    pre_m = (ai_m + aj_m + c1
             + jnp.dot(ne, wme_ref[...], preferred_element_type=jnp.float32))
    h2 = jnp.maximum(pre_m, 0.0)
    msg = jnp.dot(h2, v2_ref[...], preferred_element_type=jnp.float32) + c2
    ne_ref[...] = ne.astype(ne_ref.dtype)
    rows = i * BE + lax.broadcasted_iota(jnp.int32, (BE, 1), 0)
    msg_ref[...] = jnp.where(rows < EP, msg, 0.0)


def _edge_mlps(ai, aj, ea, we, w2e, wme, v2, bias, ne_dtype):
    d_e = ea.shape[1]
    return pl.pallas_call(
        _edge_body,
        grid=(EPAD // BE,),
        in_specs=[
            pl.BlockSpec((BE, D), lambda i: (i, 0)),
            pl.BlockSpec((BE, D), lambda i: (i, 0)),
            pl.BlockSpec((BE, d_e), lambda i: (i, 0)),
            pl.BlockSpec((d_e, D), lambda i: (0, 0)),
            pl.BlockSpec((D, D), lambda i: (0, 0)),
            pl.BlockSpec((D, D), lambda i: (0, 0)),
            pl.BlockSpec((D, D), lambda i: (0, 0)),
            pl.BlockSpec((8, D), lambda i: (0, 0)),
        ],
        out_specs=[
            pl.BlockSpec((BE, D), lambda i: (i, 0)),
            pl.BlockSpec((BE, D), lambda i: (i, 0)),
        ],
        out_shape=[
            jax.ShapeDtypeStruct((EPAD, D), ne_dtype),
            jax.ShapeDtypeStruct((EPAD, D), jnp.float32),
        ],
    )(ai, aj, ea, we, w2e, wme, v2, bias)


def _node_body_tables(x_ref, p_ref, u1x_ref, u1a_ref, u2_ref, bias_ref,
                      wd_ref, ws_ref, xo_ref, td_ref, ts_ref):
    aggr = p_ref[0] + p_ref[1]
    d1 = bias_ref[0:1, :]
    d2 = bias_ref[1:2, :]
    pre = (jnp.dot(x_ref[...], u1x_ref[...], preferred_element_type=jnp.float32)
           + jnp.dot(aggr, u1a_ref[...], preferred_element_type=jnp.float32)
           + d1)
    xn = jnp.dot(jnp.maximum(pre, 0.0), u2_ref[...],
                 preferred_element_type=jnp.float32) + d2
    xo_ref[...] = xn
    td = jnp.dot(xn, wd_ref[...], preferred_element_type=jnp.float32)
    ts = jnp.dot(xn, ws_ref[...], preferred_element_type=jnp.float32)
    td_ref[...] = _pack2(td[:, :D], td[:, D:])
    ts_ref[...] = _pack2(ts[:, :D], ts[:, D:])


def _node_body(x_ref, p_ref, u1x_ref, u1a_ref, u2_ref, bias_ref, xo_ref):
    aggr = p_ref[0] + p_ref[1]
    d1 = bias_ref[0:1, :]
    d2 = bias_ref[1:2, :]
    pre = (jnp.dot(x_ref[...], u1x_ref[...], preferred_element_type=jnp.float32)
           + jnp.dot(aggr, u1a_ref[...], preferred_element_type=jnp.float32)
           + d1)
    xo_ref[...] = jnp.dot(jnp.maximum(pre, 0.0), u2_ref[...],
                          preferred_element_type=jnp.float32) + d2


def _node_update(x, partials, u1x, u1a, u2, bias, wd=None, ws=None):
    mat = lambda i: (0, 0)
    in_specs = [
        pl.BlockSpec((BN, D), lambda i: (i, 0)),
        pl.BlockSpec((2, BN, D), lambda i: (0, i, 0)),
        pl.BlockSpec((D, D), mat),
        pl.BlockSpec((D, D), mat),
        pl.BlockSpec((D, D), mat),
        pl.BlockSpec((8, D), mat),
    ]
    if wd is None:
        return pl.pallas_call(
            _node_body,
            grid=(N // BN,),
            in_specs=in_specs,
            out_specs=pl.BlockSpec((BN, D), lambda i: (i, 0)),
            out_shape=jax.ShapeDtypeStruct((N, D), jnp.float32),
        )(x, partials, u1x, u1a, u2, bias)
    in_specs += [pl.BlockSpec((D, TWO_D), mat), pl.BlockSpec((D, TWO_D), mat)]
    return pl.pallas_call(
        _node_body_tables,
        grid=(N // BN,),
        in_specs=in_specs,
        out_specs=[
            pl.BlockSpec((BN, D), lambda i: (i, 0)),
            pl.BlockSpec((BN, D), lambda i: (i, 0)),
            pl.BlockSpec((BN, D), lambda i: (i, 0)),
        ],
        out_shape=[
            jax.ShapeDtypeStruct((N, D), jnp.float32),
            jax.ShapeDtypeStruct((N, D), jnp.int32),
            jax.ShapeDtypeStruct((N, D), jnp.int32),
        ],
    )(x, partials, u1x, u1a, u2, bias, wd, ws)


def _gather_kernel():
    mesh = plsc.VectorSubcoreMesh(core_axis_name="c", subcore_axis_name="s")

    def body(td_hbm, ts_hbm, dst_hbm, src_hbm, ai_hbm, aj_hbm,
             idxd, idxs, rowsd0, rowss0, rowsd1, rowss1,
             gd0, gs0, gd1, gs1):
        wid = lax.axis_index("s") * 2 + lax.axis_index("c")
        base = wid * EPW

        pltpu.sync_copy(dst_hbm.at[wid], idxd)
        pltpu.sync_copy(src_hbm.at[wid], idxs)

        def start(c, rd, rs, sd, ss):
            pltpu.async_copy(td_hbm.at[idxd.at[c]], rd, sd)
            pltpu.async_copy(ts_hbm.at[idxs.at[c]], rs, ss)

        def drain(c, rd, rs, sd, ss):
            pltpu.make_async_copy(td_hbm.at[idxd.at[c]], rd, sd).wait()
            pltpu.make_async_copy(ts_hbm.at[idxs.at[c]], rs, ss).wait()
            eb = base + c * CHUNK
            pltpu.sync_copy(rd, ai_hbm.at[pl.ds(eb, CHUNK)])
            pltpu.sync_copy(rs, aj_hbm.at[pl.ds(eb, CHUNK)])

        start(0, rowsd0, rowss0, gd0, gs0)

        @pl.loop(0, NCHUNK // 2)
        def _(g):
            c = 2 * g
            start(c + 1, rowsd1, rowss1, gd1, gs1)
            drain(c, rowsd0, rowss0, gd0, gs0)

            @pl.when(c + 2 < NCHUNK)
            def _():
                start(c + 2, rowsd0, rowss0, gd0, gs0)

            drain(c + 1, rowsd1, rowss1, gd1, gs1)

    return pl.kernel(
        body,
        out_type=[jax.ShapeDtypeStruct((EPAD, D), jnp.int32)] * 2,
        mesh=mesh,
        scratch_types=[
            pltpu.VMEM((NCHUNK, CHUNK), jnp.int32),
            pltpu.VMEM((NCHUNK, CHUNK), jnp.int32),
            pltpu.VMEM((CHUNK, D), jnp.int32),
            pltpu.VMEM((CHUNK, D), jnp.int32),
            pltpu.VMEM((CHUNK, D), jnp.int32),
            pltpu.VMEM((CHUNK, D), jnp.int32),
            pltpu.SemaphoreType.DMA,
            pltpu.SemaphoreType.DMA,
            pltpu.SemaphoreType.DMA,
            pltpu.SemaphoreType.DMA,
        ],
    )


def _scatter_kernel():
    mesh = plsc.VectorSubcoreMesh(core_axis_name="c", subcore_axis_name="s")

    def body(msg_hbm, dst_hbm, out_hbm, idx, mbuf0, mbuf1, acc, ls0, ls1):
        cid = lax.axis_index("c")
        sid = lax.axis_index("s")
        wid = sid * 2 + cid
        base = wid * EPW

        pltpu.sync_copy(dst_hbm.at[wid], idx)

        @pl.loop(0, CHUNK)
        def _(i):
            for j in range(D // 16):
                mbuf0[i, pl.ds(j * 16, 16)] = jnp.zeros((16,), jnp.float32)

        rbase = sid * NTS
        for k in range(NTS // CHUNK):
            pltpu.sync_copy(mbuf0, acc.at[pl.ds(rbase + k * CHUNK, CHUNK)])
        rem = NTS - (NTS // CHUNK) * CHUNK
        if rem:
            pltpu.sync_copy(mbuf0.at[pl.ds(0, rem)],
                            acc.at[pl.ds(rbase + NTS - rem, rem)])
        plsc.subcore_barrier()

        def start(c, mb, sem):
            pltpu.async_copy(msg_hbm.at[pl.ds(base + c * CHUNK, CHUNK)],
                             mb, sem)

        def drain(c, mb, sem):
            pltpu.make_async_copy(msg_hbm.at[pl.ds(base + c * CHUNK, CHUNK)],
                                  mb, sem).wait()
            pltpu.sync_copy(mb, acc.at[idx.at[c]], add=True)

        start(0, mbuf0, ls0)

        @pl.loop(0, NCHUNK // 2)
        def _(g):
            c = 2 * g
            start(c + 1, mbuf1, ls1)
            drain(c, mbuf0, ls0)

            @pl.when(c + 2 < NCHUNK)
            def _():
                start(c + 2, mbuf0, ls0)

            drain(c + 1, mbuf1, ls1)

        plsc.subcore_barrier()
        pltpu.sync_copy(acc.at[pl.ds(rbase, NTS)],
                        out_hbm.at[cid, pl.ds(rbase, NTS)])

    return pl.kernel(
        body,
        out_type=jax.ShapeDtypeStruct((2, NACC, D), jnp.float32),
        mesh=mesh,
        scratch_types=[
            pltpu.VMEM((NCHUNK, CHUNK), jnp.int32),
            pltpu.VMEM((CHUNK, D), jnp.float32),
            pltpu.VMEM((CHUNK, D), jnp.float32),
            pltpu.VMEM_SHARED((NACC, D), jnp.float32),
            pltpu.SemaphoreType.DMA,
            pltpu.SemaphoreType.DMA,
        ],
    )


def _prep_layer(p):
    en, mm = p["en"], p["mm"]
    wd = jnp.concatenate([en["W1"][:D], mm["W1"][:D]], axis=1)
    ws = jnp.concatenate([en["W1"][D:TWO_D], mm["W1"][D:TWO_D]], axis=1)
    we = en["W1"][TWO_D:]
    wme = mm["W1"][TWO_D:]
    bias = (jnp.zeros((8, D), jnp.float32)
            .at[0].set(en["b1"]).at[1].set(en["b2"])
            .at[2].set(mm["b1"]).at[3].set(mm["b2"]))
    return wd, ws, we, en["W2"], wme, mm["W2"], bias


def _prep_node(p):
    nu = p["nu"]
    bias = (jnp.zeros((8, D), jnp.float32)
            .at[0].set(nu["b1"]).at[1].set(nu["b2"]))
    return nu["W1"][:D], nu["W1"][D:], nu["W2"], bias


@jax.jit
def kernel(x, edge_attr, edge_index, params):
    ei = edge_index.astype(jnp.int32)
    loops = jnp.arange(N, dtype=jnp.int32)
    pad = jnp.zeros((EPAD - EP,), jnp.int32)
    src = jnp.concatenate([ei[0], loops, pad]).reshape(NW, NCHUNK, CHUNK)
    dst = jnp.concatenate([ei[1], loops, pad]).reshape(NW, NCHUNK, CHUNK)
    ea0 = jnp.concatenate(
        [edge_attr, jnp.zeros((EPAD - E, edge_attr.shape[1]), jnp.float32)])

    wd0, ws0, we0, w2e0, wme0, v20, be0 = _prep_layer(params["l0"])
    wd1, ws1, we1, w2e1, wme1, v21, be1 = _prep_layer(params["l1"])
    u1x0, u1a0, u20, bn0 = _prep_node(params["l0"])
    u1x1, u1a1, u21, bn1 = _prep_node(params["l1"])

    gather = _gather_kernel()
    scatter = _scatter_kernel()

    td0, ts0 = _tables(x, wd0, ws0)
    ai, aj = gather(td0, ts0, dst, src)
    ne0, msg0 = _edge_mlps(ai, aj, ea0, we0, w2e0, wme0, v20, be0,
                           jnp.bfloat16)
    part0 = scatter(msg0, dst)
    x1, td1, ts1 = _node_update(x, part0, u1x0, u1a0, u20, bn0, wd1, ws1)

    ai, aj = gather(td1, ts1, dst, src)
    ne1, msg1 = _edge_mlps(ai, aj, ne0, we1.astype(jnp.bfloat16), w2e1,
                           wme1, v21, be1, jnp.float32)
    part1 = scatter(msg1, dst)
    x2 = _node_update(x1, part1, u1x1, u1a1, u21, bn1)

    return (x2, ne1[:EP])


# speedup vs baseline: 2.6410x; 1.0505x over previous
import functools

import jax
import jax.numpy as jnp
from jax import lax
from jax.experimental import pallas as pl
from jax.experimental.pallas import tpu as pltpu
from jax.experimental.pallas import tpu_sc as plsc

N = 10000
E = 160000
D = 128
TWO_D = 2 * D
EP = E + N
NW = 32
CHUNK = 128
EPW = 5376
EPAD = EPW * NW
CH0, CH1 = 22, 20
EH0 = NW * CH0 * CHUNK
EH1 = NW * CH1 * CHUNK
NACC = 10240
NTS = NACC // 16
BE = 512
BN = 1000


def _pack2(a, b):
    ai = lax.bitcast_convert_type(
        a.astype(jnp.bfloat16).astype(jnp.float32), jnp.int32)
    bi = lax.bitcast_convert_type(
        b.astype(jnp.bfloat16).astype(jnp.float32), jnp.int32)
    return lax.shift_right_logical(ai, 16) | bi


def _unpack2(p):
    lo = lax.bitcast_convert_type(lax.shift_left(p, 16), jnp.float32)
    hi = lax.bitcast_convert_type(p & jnp.int32(-65536), jnp.float32)
    return (lo, hi)


def _tables_body(x_ref, wd_ref, ws_ref, td_ref, ts_ref):
    xb = x_ref[...]
    td = jnp.dot(xb, wd_ref[...], preferred_element_type=jnp.float32)
    ts = jnp.dot(xb, ws_ref[...], preferred_element_type=jnp.float32)
    td_ref[...] = _pack2(td[:, :D], td[:, D:])
    ts_ref[...] = _pack2(ts[:, :D], ts[:, D:])


def _tables(x, wd, ws):
    return pl.pallas_call(
        _tables_body,
        grid=(N // BN,),
        in_specs=[
            pl.BlockSpec((BN, D), lambda i: (i, 0)),
            pl.BlockSpec((D, TWO_D), lambda i: (0, 0)),
            pl.BlockSpec((D, TWO_D), lambda i: (0, 0)),
        ],
        out_specs=[
            pl.BlockSpec((BN, D), lambda i: (i, 0)),
            pl.BlockSpec((BN, D), lambda i: (i, 0)),
        ],
        out_shape=[jax.ShapeDtypeStruct((N, D), jnp.int32)] * 2,
    )(x, wd, ws)


def _edge_body(ai_ref, aj_ref, ea_ref, we_ref, w2e_ref, wme_ref, v2_ref,
               bias_ref, ne_ref, msg_ref, *, row_off):
    i = pl.program_id(0)
    b1e = bias_ref[0:1, :]
    b2e = bias_ref[1:2, :]
    c1 = bias_ref[2:3, :]
    c2 = bias_ref[3:4, :]
    ai_e, ai_m = _unpack2(ai_ref[...])
    aj_e, aj_m = _unpack2(aj_ref[...])
    pre_e = (ai_e + aj_e + b1e
             + jnp.dot(ea_ref[...], we_ref[...],
                       preferred_element_type=jnp.float32))
    h = jnp.maximum(pre_e, 0.0)
    ne = jnp.dot(h, w2e_ref[...], preferred_element_type=jnp.float32) + b2e
    pre_m = (ai_m + aj_m + c1
             + jnp.dot(ne, wme_ref[...], preferred_element_type=jnp.float32))
    h2 = jnp.maximum(pre_m, 0.0)
    msg = jnp.dot(h2, v2_ref[...], preferred_element_type=jnp.float32) + c2
    ne_ref[...] = ne.astype(ne_ref.dtype)
    rows = row_off + i * BE + lax.broadcasted_iota(jnp.int32, (BE, 1), 0)
    msg_ref[...] = jnp.where(rows < EP, msg, 0.0)


def _edge_mlps(ai, aj, ea, we, w2e, wme, v2, bias, ne_dtype, row_off):
    d_e = ea.shape[1]
    ne = ai.shape[0]
    return pl.pallas_call(
        functools.partial(_edge_body, row_off=row_off),
        grid=(ne // BE,),
        in_specs=[
            pl.BlockSpec((BE, D), lambda i: (i, 0)),
            pl.BlockSpec((BE, D), lambda i: (i, 0)),
            pl.BlockSpec((BE, d_e), lambda i: (i, 0)),
            pl.BlockSpec((d_e, D), lambda i: (0, 0)),
            pl.BlockSpec((D, D), lambda i: (0, 0)),
            pl.BlockSpec((D, D), lambda i: (0, 0)),
            pl.BlockSpec((D, D), lambda i: (0, 0)),
            pl.BlockSpec((8, D), lambda i: (0, 0)),
        ],
        out_specs=[
            pl.BlockSpec((BE, D), lambda i: (i, 0)),
            pl.BlockSpec((BE, D), lambda i: (i, 0)),
        ],
        out_shape=[
            jax.ShapeDtypeStruct((ne, D), ne_dtype),
            jax.ShapeDtypeStruct((ne, D), jnp.float32),
        ],
    )(ai, aj, ea, we, w2e, wme, v2, bias)


def _node_body_tables(x_ref, p_ref, u1x_ref, u1a_ref, u2_ref, bias_ref,
                      wd_ref, ws_ref, xo_ref, td_ref, ts_ref):
    aggr = p_ref[0] + p_ref[1]
    d1 = bias_ref[0:1, :]
    d2 = bias_ref[1:2, :]
    pre = (jnp.dot(x_ref[...], u1x_ref[...], preferred_element_type=jnp.float32)
           + jnp.dot(aggr, u1a_ref[...], preferred_element_type=jnp.float32)
           + d1)
    xn = jnp.dot(jnp.maximum(pre, 0.0), u2_ref[...],
                 preferred_element_type=jnp.float32) + d2
    xo_ref[...] = xn
    td = jnp.dot(xn, wd_ref[...], preferred_element_type=jnp.float32)
    ts = jnp.dot(xn, ws_ref[...], preferred_element_type=jnp.float32)
    td_ref[...] = _pack2(td[:, :D], td[:, D:])
    ts_ref[...] = _pack2(ts[:, :D], ts[:, D:])


def _node_body(x_ref, p_ref, u1x_ref, u1a_ref, u2_ref, bias_ref, xo_ref):
    aggr = p_ref[0] + p_ref[1]
    d1 = bias_ref[0:1, :]
    d2 = bias_ref[1:2, :]
    pre = (jnp.dot(x_ref[...], u1x_ref[...], preferred_element_type=jnp.float32)
           + jnp.dot(aggr, u1a_ref[...], preferred_element_type=jnp.float32)
           + d1)
    xo_ref[...] = jnp.dot(jnp.maximum(pre, 0.0), u2_ref[...],
                          preferred_element_type=jnp.float32) + d2


def _node_update(x, partials, u1x, u1a, u2, bias, wd=None, ws=None):
    mat = lambda i: (0, 0)
    in_specs = [
        pl.BlockSpec((BN, D), lambda i: (i, 0)),
        pl.BlockSpec((2, BN, D), lambda i: (0, i, 0)),
        pl.BlockSpec((D, D), mat),
        pl.BlockSpec((D, D), mat),
        pl.BlockSpec((D, D), mat),
        pl.BlockSpec((8, D), mat),
    ]
    if wd is None:
        return pl.pallas_call(
            _node_body,
            grid=(N // BN,),
            in_specs=in_specs,
            out_specs=pl.BlockSpec((BN, D), lambda i: (i, 0)),
            out_shape=jax.ShapeDtypeStruct((N, D), jnp.float32),
        )(x, partials, u1x, u1a, u2, bias)
    in_specs += [pl.BlockSpec((D, TWO_D), mat), pl.BlockSpec((D, TWO_D), mat)]
    return pl.pallas_call(
        _node_body_tables,
        grid=(N // BN,),
        in_specs=in_specs,
        out_specs=[
            pl.BlockSpec((BN, D), lambda i: (i, 0)),
            pl.BlockSpec((BN, D), lambda i: (i, 0)),
            pl.BlockSpec((BN, D), lambda i: (i, 0)),
        ],
        out_shape=[
            jax.ShapeDtypeStruct((N, D), jnp.float32),
            jax.ShapeDtypeStruct((N, D), jnp.int32),
            jax.ShapeDtypeStruct((N, D), jnp.int32),
        ],
    )(x, partials, u1x, u1a, u2, bias, wd, ws)


def _gather_kernel(nch):
    mesh = plsc.VectorSubcoreMesh(core_axis_name="c", subcore_axis_name="s")
    epw = nch * CHUNK

    def body(td_hbm, ts_hbm, dst_hbm, src_hbm, ai_hbm, aj_hbm,
             idxd, idxs, rowsd0, rowss0, rowsd1, rowss1,
             gd0, gs0, gd1, gs1):
        wid = lax.axis_index("s") * 2 + lax.axis_index("c")
        base = wid * epw

        pltpu.sync_copy(dst_hbm.at[wid], idxd)
        pltpu.sync_copy(src_hbm.at[wid], idxs)

        def start(c, rd, rs, sd, ss):
            pltpu.async_copy(td_hbm.at[idxd.at[c]], rd, sd)
            pltpu.async_copy(ts_hbm.at[idxs.at[c]], rs, ss)

        def drain(c, rd, rs, sd, ss):
            pltpu.make_async_copy(td_hbm.at[idxd.at[c]], rd, sd).wait()
            pltpu.make_async_copy(ts_hbm.at[idxs.at[c]], rs, ss).wait()
            eb = base + c * CHUNK
            pltpu.sync_copy(rd, ai_hbm.at[pl.ds(eb, CHUNK)])
            pltpu.sync_copy(rs, aj_hbm.at[pl.ds(eb, CHUNK)])

        start(0, rowsd0, rowss0, gd0, gs0)

        @pl.loop(0, nch // 2)
        def _(g):
            c = 2 * g
            start(c + 1, rowsd1, rowss1, gd1, gs1)
            drain(c, rowsd0, rowss0, gd0, gs0)

            @pl.when(c + 2 < nch)
            def _():
                start(c + 2, rowsd0, rowss0, gd0, gs0)

            drain(c + 1, rowsd1, rowss1, gd1, gs1)

    return pl.kernel(
        body,
        out_type=[jax.ShapeDtypeStruct((NW * epw, D), jnp.int32)] * 2,
        mesh=mesh,
        scratch_types=[
            pltpu.VMEM((nch, CHUNK), jnp.int32),
            pltpu.VMEM((nch, CHUNK), jnp.int32),
            pltpu.VMEM((CHUNK, D), jnp.int32),
            pltpu.VMEM((CHUNK, D), jnp.int32),
            pltpu.VMEM((CHUNK, D), jnp.int32),
            pltpu.VMEM((CHUNK, D), jnp.int32),
            pltpu.SemaphoreType.DMA,
            pltpu.SemaphoreType.DMA,
            pltpu.SemaphoreType.DMA,
            pltpu.SemaphoreType.DMA,
        ],
    )


def _scatter_kernel():
    mesh = plsc.VectorSubcoreMesh(core_axis_name="c", subcore_axis_name="s")

    def body(msg0_hbm, msg1_hbm, dst0_hbm, dst1_hbm, out_hbm,
             idx0, idx1, mbuf0, mbuf1, acc, ls0, ls1):
        cid = lax.axis_index("c")
        sid = lax.axis_index("s")
        wid = sid * 2 + cid

        pltpu.sync_copy(dst0_hbm.at[wid], idx0)
        pltpu.sync_copy(dst1_hbm.at[wid], idx1)

        @pl.loop(0, CHUNK)
        def _(i):
            for j in range(D // 16):
                mbuf0[i, pl.ds(j * 16, 16)] = jnp.zeros((16,), jnp.float32)

        rbase = sid * NTS
        for k in range(NTS // CHUNK):
            pltpu.sync_copy(mbuf0, acc.at[pl.ds(rbase + k * CHUNK, CHUNK)])
        plsc.subcore_barrier()

        def phase(msg_hbm, idx, nch):
            base = wid * nch * CHUNK

            def start(c, mb, sem):
                pltpu.async_copy(msg_hbm.at[pl.ds(base + c * CHUNK, CHUNK)],
                                 mb, sem)

            def drain(c, mb, sem):
                pltpu.make_async_copy(
                    msg_hbm.at[pl.ds(base + c * CHUNK, CHUNK)],
                    mb, sem).wait()
                pltpu.sync_copy(mb, acc.at[idx.at[c]], add=True)

            start(0, mbuf0, ls0)

            @pl.loop(0, nch // 2)
            def _(g):
                c = 2 * g
                start(c + 1, mbuf1, ls1)
                drain(c, mbuf0, ls0)

                @pl.when(c + 2 < nch)
                def _():
                    start(c + 2, mbuf0, ls0)

                drain(c + 1, mbuf1, ls1)

        phase(msg0_hbm, idx0, CH0)
        phase(msg1_hbm, idx1, CH1)

        plsc.subcore_barrier()
        pltpu.sync_copy(acc.at[pl.ds(rbase, NTS)],
                        out_hbm.at[cid, pl.ds(rbase, NTS)])

    return pl.kernel(
        body,
        out_type=jax.ShapeDtypeStruct((2, NACC, D), jnp.float32),
        mesh=mesh,
        scratch_types=[
            pltpu.VMEM((CH0, CHUNK), jnp.int32),
            pltpu.VMEM((CH1, CHUNK), jnp.int32),
            pltpu.VMEM((CHUNK, D), jnp.float32),
            pltpu.VMEM((CHUNK, D), jnp.float32),
            pltpu.VMEM_SHARED((NACC, D), jnp.float32),
            pltpu.SemaphoreType.DMA,
            pltpu.SemaphoreType.DMA,
        ],
    )


def _prep_layer(p):
    en, mm = p["en"], p["mm"]
    wd = jnp.concatenate([en["W1"][:D], mm["W1"][:D]], axis=1)
    ws = jnp.concatenate([en["W1"][D:TWO_D], mm["W1"][D:TWO_D]], axis=1)
    we = en["W1"][TWO_D:]
    wme = mm["W1"][TWO_D:]
    bias = (jnp.zeros((8, D), jnp.float32)
            .at[0].set(en["b1"]).at[1].set(en["b2"])
            .at[2].set(mm["b1"]).at[3].set(mm["b2"]))
    return wd, ws, we, en["W2"], wme, mm["W2"], bias


def _prep_node(p):
    nu = p["nu"]
    bias = (jnp.zeros((8, D), jnp.float32)
            .at[0].set(nu["b1"]).at[1].set(nu["b2"]))
    return nu["W1"][:D], nu["W1"][D:], nu["W2"], bias


@jax.jit
def kernel(x, edge_attr, edge_index, params):
    ei = edge_index.astype(jnp.int32)
    loops = jnp.arange(N, dtype=jnp.int32)
    pad = jnp.zeros((EPAD - EP,), jnp.int32)
    src = jnp.concatenate([ei[0], loops, pad])
    dst = jnp.concatenate([ei[1], loops, pad])
    src0 = src[:EH0].reshape(NW, CH0, CHUNK)
    src1 = src[EH0:].reshape(NW, CH1, CHUNK)
    dst0 = dst[:EH0].reshape(NW, CH0, CHUNK)
    dst1 = dst[EH0:].reshape(NW, CH1, CHUNK)
    ea_f = jnp.concatenate(
        [edge_attr, jnp.zeros((EPAD - E, edge_attr.shape[1]), jnp.float32)])
    ea_h0, ea_h1 = ea_f[:EH0], ea_f[EH0:]

    wd0, ws0, we0, w2e0, wme0, v20, be0 = _prep_layer(params["l0"])
    wd1, ws1, we1, w2e1, wme1, v21, be1 = _prep_layer(params["l1"])
    u1x0, u1a0, u20, bn0 = _prep_node(params["l0"])
    u1x1, u1a1, u21, bn1 = _prep_node(params["l1"])

    gather0 = _gather_kernel(CH0)
    gather1 = _gather_kernel(CH1)
    scatter = _scatter_kernel()

    td0, ts0 = _tables(x, wd0, ws0)
    a0i, a0j = gather0(td0, ts0, dst0, src0)
    a1i, a1j = gather1(td0, ts0, dst1, src1)
    ne00, msg00 = _edge_mlps(a0i, a0j, ea_h0, we0, w2e0, wme0, v20, be0,
                             jnp.bfloat16, 0)
    ne01, msg01 = _edge_mlps(a1i, a1j, ea_h1, we0, w2e0, wme0, v20, be0,
                             jnp.bfloat16, EH0)
    part0 = scatter(msg00, msg01, dst0, dst1)
    x1, td1, ts1 = _node_update(x, part0, u1x0, u1a0, u20, bn0, wd1, ws1)

    we1b = we1.astype(jnp.bfloat16)
    b0i, b0j = gather0(td1, ts1, dst0, src0)
    b1i, b1j = gather1(td1, ts1, dst1, src1)
    ne10, msg10 = _edge_mlps(b0i, b0j, ne00, we1b, w2e1, wme1, v21, be1,
                             jnp.float32, 0)
    ne11, msg11 = _edge_mlps(b1i, b1j, ne01, we1b, w2e1, wme1, v21, be1,
                             jnp.float32, EH0)
    part1 = scatter(msg10, msg11, dst0, dst1)
    x2 = _node_update(x1, part1, u1x1, u1a1, u21, bn1)

    return (x2, jnp.concatenate([ne10, ne11])[:EP])


# speedup vs baseline: 2.8552x; 1.0811x over previous
import functools

import jax
import jax.numpy as jnp
from jax import lax
from jax.experimental import pallas as pl
from jax.experimental.pallas import tpu as pltpu
from jax.experimental.pallas import tpu_sc as plsc

N = 10000
E = 160000
D = 128
TWO_D = 2 * D
EP = E + N
NW = 32
CHUNK = 128
EPW = 5376
EPAD = EPW * NW
PH_CH = (8, 18, 16)
PH_EH = tuple(NW * c * CHUNK for c in PH_CH)
PH_OFF = (0, PH_EH[0], PH_EH[0] + PH_EH[1])
NACC = 10240
NTS = NACC // 16
BE = 512
BN = 1000


def _pack2(a, b):
    ai = lax.bitcast_convert_type(
        a.astype(jnp.bfloat16).astype(jnp.float32), jnp.int32)
    bi = lax.bitcast_convert_type(
        b.astype(jnp.bfloat16).astype(jnp.float32), jnp.int32)
    return lax.shift_right_logical(ai, 16) | bi


def _unpack2(p):
    lo = lax.bitcast_convert_type(lax.shift_left(p, 16), jnp.float32)
    hi = lax.bitcast_convert_type(p & jnp.int32(-65536), jnp.float32)
    return (lo, hi)


def _tables_body(x_ref, wd_ref, ws_ref, td_ref, ts_ref):
    xb = x_ref[...]
    td = jnp.dot(xb, wd_ref[...], preferred_element_type=jnp.float32)
    ts = jnp.dot(xb, ws_ref[...], preferred_element_type=jnp.float32)
    td_ref[...] = _pack2(td[:, :D], td[:, D:])
    ts_ref[...] = _pack2(ts[:, :D], ts[:, D:])


def _tables(x, wd, ws):
    return pl.pallas_call(
        _tables_body,
        grid=(N // BN,),
        in_specs=[
            pl.BlockSpec((BN, D), lambda i: (i, 0)),
            pl.BlockSpec((D, TWO_D), lambda i: (0, 0)),
            pl.BlockSpec((D, TWO_D), lambda i: (0, 0)),
        ],
        out_specs=[
            pl.BlockSpec((BN, D), lambda i: (i, 0)),
            pl.BlockSpec((BN, D), lambda i: (i, 0)),
        ],
        out_shape=[jax.ShapeDtypeStruct((N, D), jnp.int32)] * 2,
    )(x, wd, ws)


def _edge_body(ai_ref, aj_ref, ea_ref, we_ref, w2e_ref, wme_ref, v2_ref,
               bias_ref, ne_ref, msg_ref, *, row_off):
    i = pl.program_id(0)
    b1e = bias_ref[0:1, :]
    b2e = bias_ref[1:2, :]
    c1 = bias_ref[2:3, :]
    c2 = bias_ref[3:4, :]
    ai_e, ai_m = _unpack2(ai_ref[...])
    aj_e, aj_m = _unpack2(aj_ref[...])
    pre_e = (ai_e + aj_e + b1e
             + jnp.dot(ea_ref[...], we_ref[...],
                       preferred_element_type=jnp.float32))
    h = jnp.maximum(pre_e, 0.0)
    ne = jnp.dot(h, w2e_ref[...], preferred_element_type=jnp.float32) + b2e
    pre_m = (ai_m + aj_m + c1
             + jnp.dot(ne, wme_ref[...], preferred_element_type=jnp.float32))
    h2 = jnp.maximum(pre_m, 0.0)
    msg = jnp.dot(h2, v2_ref[...], preferred_element_type=jnp.float32) + c2
    ne_ref[...] = ne.astype(ne_ref.dtype)
    rows = row_off + i * BE + lax.broadcasted_iota(jnp.int32, (BE, 1), 0)
    msg_ref[...] = jnp.where(rows < EP, msg, 0.0)


def _edge_mlps(ai, aj, ea, we, w2e, wme, v2, bias, ne_dtype, row_off):
    d_e = ea.shape[1]
    ne = ai.shape[0]
    return pl.pallas_call(
        functools.partial(_edge_body, row_off=row_off),
        grid=(ne // BE,),
        in_specs=[
            pl.BlockSpec((BE, D), lambda i: (i, 0)),
            pl.BlockSpec((BE, D), lambda i: (i, 0)),
            pl.BlockSpec((BE, d_e), lambda i: (i, 0)),
            pl.BlockSpec((d_e, D), lambda i: (0, 0)),
            pl.BlockSpec((D, D), lambda i: (0, 0)),
            pl.BlockSpec((D, D), lambda i: (0, 0)),
            pl.BlockSpec((D, D), lambda i: (0, 0)),
            pl.BlockSpec((8, D), lambda i: (0, 0)),
        ],
        out_specs=[
            pl.BlockSpec((BE, D), lambda i: (i, 0)),
            pl.BlockSpec((BE, D), lambda i: (i, 0)),
        ],
        out_shape=[
            jax.ShapeDtypeStruct((ne, D), ne_dtype),
            jax.ShapeDtypeStruct((ne, D), jnp.float32),
        ],
    )(ai, aj, ea, we, w2e, wme, v2, bias)


def _node_body_tables(x_ref, p_ref, u1x_ref, u1a_ref, u2_ref, bias_ref,
                      wd_ref, ws_ref, xo_ref, td_ref, ts_ref):
    aggr = p_ref[0] + p_ref[1]
    d1 = bias_ref[0:1, :]
    d2 = bias_ref[1:2, :]
    pre = (jnp.dot(x_ref[...], u1x_ref[...], preferred_element_type=jnp.float32)
           + jnp.dot(aggr, u1a_ref[...], preferred_element_type=jnp.float32)
           + d1)
    xn = jnp.dot(jnp.maximum(pre, 0.0), u2_ref[...],
                 preferred_element_type=jnp.float32) + d2
    xo_ref[...] = xn
    td = jnp.dot(xn, wd_ref[...], preferred_element_type=jnp.float32)
    ts = jnp.dot(xn, ws_ref[...], preferred_element_type=jnp.float32)
    td_ref[...] = _pack2(td[:, :D], td[:, D:])
    ts_ref[...] = _pack2(ts[:, :D], ts[:, D:])


def _node_body(x_ref, p_ref, u1x_ref, u1a_ref, u2_ref, bias_ref, xo_ref):
    aggr = p_ref[0] + p_ref[1]
    d1 = bias_ref[0:1, :]
    d2 = bias_ref[1:2, :]
    pre = (jnp.dot(x_ref[...], u1x_ref[...], preferred_element_type=jnp.float32)
           + jnp.dot(aggr, u1a_ref[...], preferred_element_type=jnp.float32)
           + d1)
    xo_ref[...] = jnp.dot(jnp.maximum(pre, 0.0), u2_ref[...],
                          preferred_element_type=jnp.float32) + d2


def _node_update(x, partials, u1x, u1a, u2, bias, wd=None, ws=None):
    mat = lambda i: (0, 0)
    in_specs = [
        pl.BlockSpec((BN, D), lambda i: (i, 0)),
        pl.BlockSpec((2, BN, D), lambda i: (0, i, 0)),
        pl.BlockSpec((D, D), mat),
        pl.BlockSpec((D, D), mat),
        pl.BlockSpec((D, D), mat),
        pl.BlockSpec((8, D), mat),
    ]
    if wd is None:
        return pl.pallas_call(
            _node_body,
            grid=(N // BN,),
            in_specs=in_specs,
            out_specs=pl.BlockSpec((BN, D), lambda i: (i, 0)),
            out_shape=jax.ShapeDtypeStruct((N, D), jnp.float32),
        )(x, partials, u1x, u1a, u2, bias)
    in_specs += [pl.BlockSpec((D, TWO_D), mat), pl.BlockSpec((D, TWO_D), mat)]
    return pl.pallas_call(
        _node_body_tables,
        grid=(N // BN,),
        in_specs=in_specs,
        out_specs=[
            pl.BlockSpec((BN, D), lambda i: (i, 0)),
            pl.BlockSpec((BN, D), lambda i: (i, 0)),
            pl.BlockSpec((BN, D), lambda i: (i, 0)),
        ],
        out_shape=[
            jax.ShapeDtypeStruct((N, D), jnp.float32),
            jax.ShapeDtypeStruct((N, D), jnp.int32),
            jax.ShapeDtypeStruct((N, D), jnp.int32),
        ],
    )(x, partials, u1x, u1a, u2, bias, wd, ws)


def _gather_kernel(nch):
    mesh = plsc.VectorSubcoreMesh(core_axis_name="c", subcore_axis_name="s")
    epw = nch * CHUNK

    def body(td_hbm, ts_hbm, dst_hbm, src_hbm, ai_hbm, aj_hbm,
             idxd, idxs, rowsd0, rowss0, rowsd1, rowss1,
             gd0, gs0, gd1, gs1):
        wid = lax.axis_index("s") * 2 + lax.axis_index("c")
        base = wid * epw

        pltpu.sync_copy(dst_hbm.at[wid], idxd)
        pltpu.sync_copy(src_hbm.at[wid], idxs)

        def start(c, rd, rs, sd, ss):
            pltpu.async_copy(td_hbm.at[idxd.at[c]], rd, sd)
            pltpu.async_copy(ts_hbm.at[idxs.at[c]], rs, ss)

        def drain(c, rd, rs, sd, ss):
            pltpu.make_async_copy(td_hbm.at[idxd.at[c]], rd, sd).wait()
            pltpu.make_async_copy(ts_hbm.at[idxs.at[c]], rs, ss).wait()
            eb = base + c * CHUNK
            pltpu.sync_copy(rd, ai_hbm.at[pl.ds(eb, CHUNK)])
            pltpu.sync_copy(rs, aj_hbm.at[pl.ds(eb, CHUNK)])

        start(0, rowsd0, rowss0, gd0, gs0)

        @pl.loop(0, nch // 2)
        def _(g):
            c = 2 * g
            start(c + 1, rowsd1, rowss1, gd1, gs1)
            drain(c, rowsd0, rowss0, gd0, gs0)

            @pl.when(c + 2 < nch)
            def _():
                start(c + 2, rowsd0, rowss0, gd0, gs0)

            drain(c + 1, rowsd1, rowss1, gd1, gs1)

    return pl.kernel(
        body,
        out_type=[jax.ShapeDtypeStruct((NW * epw, D), jnp.int32)] * 2,
        mesh=mesh,
        scratch_types=[
            pltpu.VMEM((nch, CHUNK), jnp.int32),
            pltpu.VMEM((nch, CHUNK), jnp.int32),
            pltpu.VMEM((CHUNK, D), jnp.int32),
            pltpu.VMEM((CHUNK, D), jnp.int32),
            pltpu.VMEM((CHUNK, D), jnp.int32),
            pltpu.VMEM((CHUNK, D), jnp.int32),
            pltpu.SemaphoreType.DMA,
            pltpu.SemaphoreType.DMA,
            pltpu.SemaphoreType.DMA,
            pltpu.SemaphoreType.DMA,
        ],
    )


def _scatter_kernel():
    mesh = plsc.VectorSubcoreMesh(core_axis_name="c", subcore_axis_name="s")

    def body(msg0_hbm, msg1_hbm, msg2_hbm, dst0_hbm, dst1_hbm, dst2_hbm,
             out_hbm, idx0, idx1, idx2, mbuf0, mbuf1, acc, ls0, ls1):
        cid = lax.axis_index("c")
        sid = lax.axis_index("s")
        wid = sid * 2 + cid

        pltpu.sync_copy(dst0_hbm.at[wid], idx0)
        pltpu.sync_copy(dst1_hbm.at[wid], idx1)
        pltpu.sync_copy(dst2_hbm.at[wid], idx2)

        @pl.loop(0, CHUNK)
        def _(i):
            for j in range(D // 16):
                mbuf0[i, pl.ds(j * 16, 16)] = jnp.zeros((16,), jnp.float32)

        rbase = sid * NTS
        for k in range(NTS // CHUNK):
            pltpu.sync_copy(mbuf0, acc.at[pl.ds(rbase + k * CHUNK, CHUNK)])
        plsc.subcore_barrier()

        def phase(msg_hbm, idx, nch):
            base = wid * nch * CHUNK

            def start(c, mb, sem):
                pltpu.async_copy(msg_hbm.at[pl.ds(base + c * CHUNK, CHUNK)],
                                 mb, sem)

            def drain(c, mb, sem):
                pltpu.make_async_copy(
                    msg_hbm.at[pl.ds(base + c * CHUNK, CHUNK)],
                    mb, sem).wait()
                pltpu.sync_copy(mb, acc.at[idx.at[c]], add=True)

            start(0, mbuf0, ls0)

            @pl.loop(0, nch // 2)
            def _(g):
                c = 2 * g
                start(c + 1, mbuf1, ls1)
                drain(c, mbuf0, ls0)

                @pl.when(c + 2 < nch)
                def _():
                    start(c + 2, mbuf0, ls0)

                drain(c + 1, mbuf1, ls1)

        phase(msg0_hbm, idx0, PH_CH[0])
        phase(msg1_hbm, idx1, PH_CH[1])
        phase(msg2_hbm, idx2, PH_CH[2])

        plsc.subcore_barrier()
        pltpu.sync_copy(acc.at[pl.ds(rbase, NTS)],
                        out_hbm.at[cid, pl.ds(rbase, NTS)])

    return pl.kernel(
        body,
        out_type=jax.ShapeDtypeStruct((2, NACC, D), jnp.float32),
        mesh=mesh,
        scratch_types=[
            pltpu.VMEM((PH_CH[0], CHUNK), jnp.int32),
            pltpu.VMEM((PH_CH[1], CHUNK), jnp.int32),
            pltpu.VMEM((PH_CH[2], CHUNK), jnp.int32),
            pltpu.VMEM((CHUNK, D), jnp.float32),
            pltpu.VMEM((CHUNK, D), jnp.float32),
            pltpu.VMEM_SHARED((NACC, D), jnp.float32),
            pltpu.SemaphoreType.DMA,
            pltpu.SemaphoreType.DMA,
        ],
    )


def _prep_layer(p):
    en, mm = p["en"], p["mm"]
    wd = jnp.concatenate([en["W1"][:D], mm["W1"][:D]], axis=1)
    ws = jnp.concatenate([en["W1"][D:TWO_D], mm["W1"][D:TWO_D]], axis=1)
    we = en["W1"][TWO_D:]
    wme = mm["W1"][TWO_D:]
    bias = (jnp.zeros((8, D), jnp.float32)
            .at[0].set(en["b1"]).at[1].set(en["b2"])
            .at[2].set(mm["b1"]).at[3].set(mm["b2"]))
    return wd, ws, we, en["W2"], wme, mm["W2"], bias


def _prep_node(p):
    nu = p["nu"]
    bias = (jnp.zeros((8, D), jnp.float32)
            .at[0].set(nu["b1"]).at[1].set(nu["b2"]))
    return nu["W1"][:D], nu["W1"][D:], nu["W2"], bias


@jax.jit
def kernel(x, edge_attr, edge_index, params):
    ei = edge_index.astype(jnp.int32)
    loops = jnp.arange(N, dtype=jnp.int32)
    pad = jnp.zeros((EPAD - EP,), jnp.int32)
    src = jnp.concatenate([ei[0], loops, pad])
    dst = jnp.concatenate([ei[1], loops, pad])
    srcs = [src[o:o + n].reshape(NW, c, CHUNK)
            for o, n, c in zip(PH_OFF, PH_EH, PH_CH)]
    dsts = [dst[o:o + n].reshape(NW, c, CHUNK)
            for o, n, c in zip(PH_OFF, PH_EH, PH_CH)]
    ea_f = jnp.concatenate(
        [edge_attr, jnp.zeros((EPAD - E, edge_attr.shape[1]), jnp.float32)])
    eas = [ea_f[o:o + n] for o, n in zip(PH_OFF, PH_EH)]

    wd0, ws0, we0, w2e0, wme0, v20, be0 = _prep_layer(params["l0"])
    wd1, ws1, we1, w2e1, wme1, v21, be1 = _prep_layer(params["l1"])
    u1x0, u1a0, u20, bn0 = _prep_node(params["l0"])
    u1x1, u1a1, u21, bn1 = _prep_node(params["l1"])

    gathers = [_gather_kernel(c) for c in PH_CH]
    scatter = _scatter_kernel()

    td0, ts0 = _tables(x, wd0, ws0)
    ga = [g(td0, ts0, d, s) for g, d, s in zip(gathers, dsts, srcs)]
    e0 = [_edge_mlps(ai, aj, ea, we0, w2e0, wme0, v20, be0,
                     jnp.bfloat16, off)
          for (ai, aj), ea, off in zip(ga, eas, PH_OFF)]
    part0 = scatter(e0[0][1], e0[1][1], e0[2][1], *dsts)
    x1, td1, ts1 = _node_update(x, part0, u1x0, u1a0, u20, bn0, wd1, ws1)

    we1b = we1.astype(jnp.bfloat16)
    gb = [g(td1, ts1, d, s) for g, d, s in zip(gathers, dsts, srcs)]
    e1 = [_edge_mlps(ai, aj, ne, we1b, w2e1, wme1, v21, be1,
                     jnp.float32, off)
          for (ai, aj), (ne, _), off in zip(gb, e0, PH_OFF)]
    part1 = scatter(e1[0][1], e1[1][1], e1[2][1], *dsts)
    x2 = _node_update(x1, part1, u1x1, u1a1, u21, bn1)

    return (x2, jnp.concatenate([ne for ne, _ in e1])[:EP])


# speedup vs baseline: 2.9193x; 1.0224x over previous
import functools

import jax
import jax.numpy as jnp
from jax import lax
from jax.experimental import pallas as pl
from jax.experimental.pallas import tpu as pltpu
from jax.experimental.pallas import tpu_sc as plsc

N = 10000
E = 160000
D = 128
TWO_D = 2 * D
EP = E + N
NW = 32
CHUNK = 128
EPW = 5376
EPAD = EPW * NW
PH_CH = (8, 18, 16)
PH_EH = tuple(NW * c * CHUNK for c in PH_CH)
PH_OFF = (0, PH_EH[0], PH_EH[0] + PH_EH[1])
NACC = 10240
NTS = NACC // 16
BE = 512
BN = 1000


def _pack2(a, b):
    ai = lax.bitcast_convert_type(
        a.astype(jnp.bfloat16).astype(jnp.float32), jnp.int32)
    bi = lax.bitcast_convert_type(
        b.astype(jnp.bfloat16).astype(jnp.float32), jnp.int32)
    return lax.shift_right_logical(ai, 16) | bi


def _unpack2(p):
    lo = lax.bitcast_convert_type(lax.shift_left(p, 16), jnp.float32)
    hi = lax.bitcast_convert_type(p & jnp.int32(-65536), jnp.float32)
    return (lo, hi)


def _tables_body(x_ref, wd_ref, ws_ref, td_ref, ts_ref):
    xb = x_ref[...]
    td = jnp.dot(xb, wd_ref[...], preferred_element_type=jnp.float32)
    ts = jnp.dot(xb, ws_ref[...], preferred_element_type=jnp.float32)
    td_ref[...] = _pack2(td[:, :D], td[:, D:])
    ts_ref[...] = _pack2(ts[:, :D], ts[:, D:])


def _tables(x, wd, ws):
    return pl.pallas_call(
        _tables_body,
        grid=(N // BN,),
        in_specs=[
            pl.BlockSpec((BN, D), lambda i: (i, 0)),
            pl.BlockSpec((D, TWO_D), lambda i: (0, 0)),
            pl.BlockSpec((D, TWO_D), lambda i: (0, 0)),
        ],
        out_specs=[
            pl.BlockSpec((BN, D), lambda i: (i, 0)),
            pl.BlockSpec((BN, D), lambda i: (i, 0)),
        ],
        out_shape=[jax.ShapeDtypeStruct((N, D), jnp.int32)] * 2,
    )(x, wd, ws)


def _edge_body(ai_ref, aj_ref, ea_ref, we_ref, w2e_ref, wme_ref, v2_ref,
               bias_ref, ne_ref, msg_ref, *, row_off):
    i = pl.program_id(0)
    b1e = bias_ref[0:1, :]
    b2e = bias_ref[1:2, :]
    c1 = bias_ref[2:3, :]
    c2 = bias_ref[3:4, :]
    ai_e, ai_m = _unpack2(ai_ref[...])
    aj_e, aj_m = _unpack2(aj_ref[...])
    pre_e = (ai_e + aj_e + b1e
             + jnp.dot(ea_ref[...], we_ref[...],
                       preferred_element_type=jnp.float32))
    h = jnp.maximum(pre_e, 0.0)
    ne = jnp.dot(h, w2e_ref[...], preferred_element_type=jnp.float32) + b2e
    pre_m = (ai_m + aj_m + c1
             + jnp.dot(ne, wme_ref[...], preferred_element_type=jnp.float32))
    h2 = jnp.maximum(pre_m, 0.0)
    msg = jnp.dot(h2, v2_ref[...], preferred_element_type=jnp.float32) + c2
    ne_ref[...] = ne.astype(ne_ref.dtype)
    rows = row_off + i * BE + lax.broadcasted_iota(jnp.int32, (BE, 1), 0)
    msg_ref[...] = jnp.where(rows < EP, msg, 0.0)


def _edge_mlps(ai, aj, ea, we, w2e, wme, v2, bias, ne_dtype, row_off):
    d_e = ea.shape[1]
    ne = ai.shape[0]
    return pl.pallas_call(
        functools.partial(_edge_body, row_off=row_off),
        grid=(ne // BE,),
        in_specs=[
            pl.BlockSpec((BE, D), lambda i: (i, 0)),
            pl.BlockSpec((BE, D), lambda i: (i, 0)),
            pl.BlockSpec((BE, d_e), lambda i: (i, 0)),
            pl.BlockSpec((d_e, D), lambda i: (0, 0)),
            pl.BlockSpec((D, D), lambda i: (0, 0)),
            pl.BlockSpec((D, D), lambda i: (0, 0)),
            pl.BlockSpec((D, D), lambda i: (0, 0)),
            pl.BlockSpec((8, D), lambda i: (0, 0)),
        ],
        out_specs=[
            pl.BlockSpec((BE, D), lambda i: (i, 0)),
            pl.BlockSpec((BE, D), lambda i: (i, 0)),
        ],
        out_shape=[
            jax.ShapeDtypeStruct((ne, D), ne_dtype),
            jax.ShapeDtypeStruct((ne, D), jnp.float32),
        ],
    )(ai, aj, ea, we, w2e, wme, v2, bias)


def _node_body_tables(x_ref, p_ref, u1x_ref, u1a_ref, u2_ref, bias_ref,
                      wd_ref, ws_ref, xo_ref, td_ref, ts_ref):
    aggr = p_ref[0] + p_ref[1]
    d1 = bias_ref[0:1, :]
    d2 = bias_ref[1:2, :]
    pre = (jnp.dot(x_ref[...], u1x_ref[...], preferred_element_type=jnp.float32)
           + jnp.dot(aggr, u1a_ref[...], preferred_element_type=jnp.float32)
           + d1)
    xn = jnp.dot(jnp.maximum(pre, 0.0), u2_ref[...],
                 preferred_element_type=jnp.float32) + d2
    xo_ref[...] = xn
    td = jnp.dot(xn, wd_ref[...], preferred_element_type=jnp.float32)
    ts = jnp.dot(xn, ws_ref[...], preferred_element_type=jnp.float32)
    td_ref[...] = _pack2(td[:, :D], td[:, D:])
    ts_ref[...] = _pack2(ts[:, :D], ts[:, D:])


def _node_body(x_ref, p_ref, u1x_ref, u1a_ref, u2_ref, bias_ref, xo_ref):
    aggr = p_ref[0] + p_ref[1]
    d1 = bias_ref[0:1, :]
    d2 = bias_ref[1:2, :]
    pre = (jnp.dot(x_ref[...], u1x_ref[...], preferred_element_type=jnp.float32)
           + jnp.dot(aggr, u1a_ref[...], preferred_element_type=jnp.float32)
           + d1)
    xo_ref[...] = jnp.dot(jnp.maximum(pre, 0.0), u2_ref[...],
                          preferred_element_type=jnp.float32) + d2


def _node_update(x, partials, u1x, u1a, u2, bias, wd=None, ws=None):
    mat = lambda i: (0, 0)
    in_specs = [
        pl.BlockSpec((BN, D), lambda i: (i, 0)),
        pl.BlockSpec((2, BN, D), lambda i: (0, i, 0)),
        pl.BlockSpec((D, D), mat),
        pl.BlockSpec((D, D), mat),
        pl.BlockSpec((D, D), mat),
        pl.BlockSpec((8, D), mat),
    ]
    if wd is None:
        return pl.pallas_call(
            _node_body,
            grid=(N // BN,),
            in_specs=in_specs,
            out_specs=pl.BlockSpec((BN, D), lambda i: (i, 0)),
            out_shape=jax.ShapeDtypeStruct((N, D), jnp.float32),
        )(x, partials, u1x, u1a, u2, bias)
    in_specs += [pl.BlockSpec((D, TWO_D), mat), pl.BlockSpec((D, TWO_D), mat)]
    return pl.pallas_call(
        _node_body_tables,
        grid=(N // BN,),
        in_specs=in_specs,
        out_specs=[
            pl.BlockSpec((BN, D), lambda i: (i, 0)),
            pl.BlockSpec((BN, D), lambda i: (i, 0)),
            pl.BlockSpec((BN, D), lambda i: (i, 0)),
        ],
        out_shape=[
            jax.ShapeDtypeStruct((N, D), jnp.float32),
            jax.ShapeDtypeStruct((N, D), jnp.int32),
            jax.ShapeDtypeStruct((N, D), jnp.int32),
        ],
    )(x, partials, u1x, u1a, u2, bias, wd, ws)


def _gather_kernel(nch):
    mesh = plsc.VectorSubcoreMesh(core_axis_name="c", subcore_axis_name="s")
    epw = nch * CHUNK

    def body(td_hbm, ts_hbm, dst_hbm, src_hbm, ai_hbm, aj_hbm,
             idxd, idxs, rowsd0, rowss0, rowsd1, rowss1,
             gd0, gs0, gd1, gs1):
        wid = lax.axis_index("s") * 2 + lax.axis_index("c")
        base = wid * epw

        pltpu.sync_copy(dst_hbm.at[wid], idxd)
        pltpu.sync_copy(src_hbm.at[wid], idxs)

        def start(c, rd, rs, sd, ss):
            pltpu.async_copy(td_hbm.at[idxd.at[c]], rd, sd)
            pltpu.async_copy(ts_hbm.at[idxs.at[c]], rs, ss)

        def drain(c, rd, rs, sd, ss):
            pltpu.make_async_copy(td_hbm.at[idxd.at[c]], rd, sd).wait()
            pltpu.make_async_copy(ts_hbm.at[idxs.at[c]], rs, ss).wait()
            eb = base + c * CHUNK
            pltpu.sync_copy(rd, ai_hbm.at[pl.ds(eb, CHUNK)])
            pltpu.sync_copy(rs, aj_hbm.at[pl.ds(eb, CHUNK)])

        start(0, rowsd0, rowss0, gd0, gs0)

        @pl.loop(0, nch // 2)
        def _(g):
            c = 2 * g
            start(c + 1, rowsd1, rowss1, gd1, gs1)
            drain(c, rowsd0, rowss0, gd0, gs0)

            @pl.when(c + 2 < nch)
            def _():
                start(c + 2, rowsd0, rowss0, gd0, gs0)

            drain(c + 1, rowsd1, rowss1, gd1, gs1)

    return pl.kernel(
        body,
        out_type=[jax.ShapeDtypeStruct((NW * epw, D), jnp.int32)] * 2,
        mesh=mesh,
        scratch_types=[
            pltpu.VMEM((nch, CHUNK), jnp.int32),
            pltpu.VMEM((nch, CHUNK), jnp.int32),
            pltpu.VMEM((CHUNK, D), jnp.int32),
            pltpu.VMEM((CHUNK, D), jnp.int32),
            pltpu.VMEM((CHUNK, D), jnp.int32),
            pltpu.VMEM((CHUNK, D), jnp.int32),
            pltpu.SemaphoreType.DMA,
            pltpu.SemaphoreType.DMA,
            pltpu.SemaphoreType.DMA,
            pltpu.SemaphoreType.DMA,
        ],
    )


def _scatter_kernel(chs, init_from_partial):
    mesh = plsc.VectorSubcoreMesh(core_axis_name="c", subcore_axis_name="s")
    nph = len(chs)

    def body(*refs):
        msgs = refs[0:nph]
        dsts = refs[nph:2 * nph]
        k = 2 * nph
        pin_hbm = None
        if init_from_partial:
            pin_hbm = refs[k]
            k += 1
        out_hbm = refs[k]
        idxs = refs[k + 1:k + 1 + nph]
        mbuf0, mbuf1, acc, ls0, ls1 = refs[k + 1 + nph:]

        cid = lax.axis_index("c")
        sid = lax.axis_index("s")
        wid = sid * 2 + cid

        for d_hbm, idx in zip(dsts, idxs):
            pltpu.sync_copy(d_hbm.at[wid], idx)

        rbase = sid * NTS
        if init_from_partial:
            pltpu.sync_copy(pin_hbm.at[cid, pl.ds(rbase, NTS)],
                            acc.at[pl.ds(rbase, NTS)])
        else:
            @pl.loop(0, CHUNK)
            def _(i):
                for j in range(D // 16):
                    mbuf0[i, pl.ds(j * 16, 16)] = jnp.zeros((16,), jnp.float32)

            for k2 in range(NTS // CHUNK):
                pltpu.sync_copy(mbuf0, acc.at[pl.ds(rbase + k2 * CHUNK, CHUNK)])
        plsc.subcore_barrier()

        def phase(msg_hbm, idx, nch):
            base = wid * nch * CHUNK

            def start(c, mb, sem):
                pltpu.async_copy(msg_hbm.at[pl.ds(base + c * CHUNK, CHUNK)],
                                 mb, sem)

            def drain(c, mb, sem):
                pltpu.make_async_copy(
                    msg_hbm.at[pl.ds(base + c * CHUNK, CHUNK)],
                    mb, sem).wait()
                pltpu.sync_copy(mb, acc.at[idx.at[c]], add=True)

            start(0, mbuf0, ls0)

            @pl.loop(0, nch // 2)
            def _(g):
                c = 2 * g
                start(c + 1, mbuf1, ls1)
                drain(c, mbuf0, ls0)

                @pl.when(c + 2 < nch)
                def _():
                    start(c + 2, mbuf0, ls0)

                drain(c + 1, mbuf1, ls1)

        for msg_hbm, idx, nch in zip(msgs, idxs, chs):
            phase(msg_hbm, idx, nch)

        plsc.subcore_barrier()
        pltpu.sync_copy(acc.at[pl.ds(rbase, NTS)],
                        out_hbm.at[cid, pl.ds(rbase, NTS)])

    return pl.kernel(
        body,
        out_type=jax.ShapeDtypeStruct((2, NACC, D), jnp.float32),
        mesh=mesh,
        scratch_types=[
            *[pltpu.VMEM((c, CHUNK), jnp.int32) for c in chs],
            pltpu.VMEM((CHUNK, D), jnp.float32),
            pltpu.VMEM((CHUNK, D), jnp.float32),
            pltpu.VMEM_SHARED((NACC, D), jnp.float32),
            pltpu.SemaphoreType.DMA,
            pltpu.SemaphoreType.DMA,
        ],
    )


def _prep_layer(p):
    en, mm = p["en"], p["mm"]
    wd = jnp.concatenate([en["W1"][:D], mm["W1"][:D]], axis=1)
    ws = jnp.concatenate([en["W1"][D:TWO_D], mm["W1"][D:TWO_D]], axis=1)
    we = en["W1"][TWO_D:]
    wme = mm["W1"][TWO_D:]
    bias = (jnp.zeros((8, D), jnp.float32)
            .at[0].set(en["b1"]).at[1].set(en["b2"])
            .at[2].set(mm["b1"]).at[3].set(mm["b2"]))
    return wd, ws, we, en["W2"], wme, mm["W2"], bias


def _prep_node(p):
    nu = p["nu"]
    bias = (jnp.zeros((8, D), jnp.float32)
            .at[0].set(nu["b1"]).at[1].set(nu["b2"]))
    return nu["W1"][:D], nu["W1"][D:], nu["W2"], bias


@jax.jit
def kernel(x, edge_attr, edge_index, params):
    ei = edge_index.astype(jnp.int32)
    loops = jnp.arange(N, dtype=jnp.int32)
    pad = jnp.zeros((EPAD - EP,), jnp.int32)
    src = jnp.concatenate([ei[0], loops, pad])
    dst = jnp.concatenate([ei[1], loops, pad])
    srcs = [src[o:o + n].reshape(NW, c, CHUNK)
            for o, n, c in zip(PH_OFF, PH_EH, PH_CH)]
    dsts = [dst[o:o + n].reshape(NW, c, CHUNK)
            for o, n, c in zip(PH_OFF, PH_EH, PH_CH)]
    ea_f = jnp.concatenate(
        [edge_attr, jnp.zeros((EPAD - E, edge_attr.shape[1]), jnp.float32)])
    eas = [ea_f[o:o + n] for o, n in zip(PH_OFF, PH_EH)]

    wd0, ws0, we0, w2e0, wme0, v20, be0 = _prep_layer(params["l0"])
    wd1, ws1, we1, w2e1, wme1, v21, be1 = _prep_layer(params["l1"])
    u1x0, u1a0, u20, bn0 = _prep_node(params["l0"])
    u1x1, u1a1, u21, bn1 = _prep_node(params["l1"])

    gathers = [_gather_kernel(c) for c in PH_CH]
    scat_a = _scatter_kernel(PH_CH[:2], False)
    scat_b = _scatter_kernel(PH_CH[2:], True)

    def scatter(m0, m1, m2, d0, d1, d2):
        pab = scat_a(m0, m1, d0, d1)
        return scat_b(m2, d2, pab)

    td0, ts0 = _tables(x, wd0, ws0)
    ga = [g(td0, ts0, d, s) for g, d, s in zip(gathers, dsts, srcs)]
    e0 = [_edge_mlps(ai, aj, ea, we0, w2e0, wme0, v20, be0,
                     jnp.bfloat16, off)
          for (ai, aj), ea, off in zip(ga, eas, PH_OFF)]
    part0 = scatter(e0[0][1], e0[1][1], e0[2][1], *dsts)
    x1, td1, ts1 = _node_update(x, part0, u1x0, u1a0, u20, bn0, wd1, ws1)

    we1b = we1.astype(jnp.bfloat16)
    gb = [g(td1, ts1, d, s) for g, d, s in zip(gathers, dsts, srcs)]
    e1 = [_edge_mlps(ai, aj, ne, we1b, w2e1, wme1, v21, be1,
                     jnp.float32, off)
          for (ai, aj), (ne, _), off in zip(gb, e0, PH_OFF)]
    part1 = scatter(e1[0][1], e1[1][1], e1[2][1], *dsts)
    x2 = _node_update(x1, part1, u1x1, u1a1, u21, bn1)

    return (x2, jnp.concatenate([ne for ne, _ in e1])[:EP])
